# Initial kernel scaffold; baseline (speedup 1.0000x reference)
import jax
import jax.numpy as jnp
from jax.experimental import pallas as pl


def kernel(features, edge_index, W1, b1, W2, b2):
    raise NotImplementedError("write your pallas kernel here")


# speedup vs baseline: 10.0219x; 10.0219x over previous
import functools

import jax
import jax.numpy as jnp
from jax import lax
from jax.experimental import pallas as pl
from jax.experimental.pallas import tpu as pltpu
from jax.experimental.pallas import tpu_sc as plsc

NC = 2
NS = 16
NW = NC * NS
CHUNK = 125


def _mesh():
    return plsc.VectorSubcoreMesh(core_axis_name="c", subcore_axis_name="s")


def _make_degree_kernel(n, kd):
    n2 = 2 * n
    zchunk = 2000
    assert n2 % zchunk == 0

    @functools.partial(
        pl.kernel,
        out_type=jax.ShapeDtypeStruct((NC, n2), jnp.float32),
        mesh=_mesh(),
        scratch_types=[
            pltpu.VMEM((kd, CHUNK), jnp.int32),
            pltpu.VMEM((zchunk,), jnp.float32),
            pltpu.VMEM((128,), jnp.float32),
            pltpu.VMEM_SHARED((n2,), jnp.float32),
        ],
    )
    def deg_kernel(didx_hbm, out_hbm, idx_v, zbuf, ones_v, acc):
        cid = lax.axis_index("c")
        sid = lax.axis_index("s")
        wid = sid * NC + cid

        @pl.when(sid == 0)
        def _():
            def zfill(i, carry):
                zbuf[pl.ds(i * 16, 16)] = jnp.zeros((16,), jnp.float32)
                return carry
            lax.fori_loop(0, zchunk // 16, zfill, 0)
            for i in range(n2 // zchunk):
                pltpu.sync_copy(zbuf, acc.at[pl.ds(i * zchunk, zchunk)])

        def ofill(i, carry):
            ones_v[pl.ds(i * 16, 16)] = jnp.ones((16,), jnp.float32)
            return carry
        lax.fori_loop(0, 128 // 16, ofill, 0)
        pltpu.sync_copy(didx_hbm.at[wid], idx_v)
        plsc.subcore_barrier()

        def body(j, carry):
            pltpu.sync_copy(ones_v.at[pl.ds(0, CHUNK)],
                            acc.at[idx_v.at[j]], add=True)
            return carry
        lax.fori_loop(0, kd, body, 0)
        plsc.subcore_barrier()

        @pl.when(sid == 0)
        def _():
            pltpu.sync_copy(acc, out_hbm.at[cid])

    return deg_kernel


def _make_agg_kernel(n, k, f):
    n_pad = ((n + 8 * NS - 1) // (8 * NS)) * (8 * NS)
    rows_per_tile = n_pad // NS

    @functools.partial(
        pl.kernel,
        out_type=jax.ShapeDtypeStruct((NC, n_pad, f), jnp.float32),
        mesh=_mesh(),
        scratch_types=[
            pltpu.VMEM((k, CHUNK), jnp.int32),
            pltpu.VMEM((k, CHUNK), jnp.int32),
            pltpu.VMEM((2, CHUNK, f), jnp.float32),
            pltpu.VMEM((rows_per_tile, f), jnp.float32),
            pltpu.VMEM_SHARED((n_pad, f), jnp.float32),
            pltpu.SemaphoreType.DMA((2,)),
        ],
        compiler_params=pltpu.CompilerParams(use_tc_tiling_on_sc=False),
    )
    def agg_kernel(h_hbm, src_hbm, dst_hbm, out_hbm,
                   src_v, dst_v, buf, zbuf, acc, sem):
        cid = lax.axis_index("c")
        sid = lax.axis_index("s")
        wid = sid * NC + cid

        def zfill(i, carry):
            zbuf[i, :] = jnp.zeros((f,), jnp.float32)
            return carry
        lax.fori_loop(0, rows_per_tile, zfill, 0)
        pltpu.sync_copy(zbuf, acc.at[pl.ds(sid * rows_per_tile, rows_per_tile)])

        pltpu.sync_copy(src_hbm.at[wid], src_v)
        pltpu.sync_copy(dst_hbm.at[wid], dst_v)
        plsc.subcore_barrier()

        pltpu.async_copy(h_hbm.at[src_v.at[0]], buf.at[0], sem.at[0])

        def body(j, carry):
            @pl.when(j + 1 < k)
            def _():
                pltpu.async_copy(h_hbm.at[src_v.at[j + 1]],
                                 buf.at[(j + 1) % 2],
                                 sem.at[(j + 1) % 2])

            pltpu.make_async_copy(h_hbm.at[src_v.at[j]],
                                  buf.at[j % 2], sem.at[j % 2]).wait()
            pltpu.sync_copy(buf.at[j % 2], acc.at[dst_v.at[j]], add=True)
            return carry

        lax.fori_loop(0, k, body, 0)
        plsc.subcore_barrier()

        pltpu.sync_copy(
            acc.at[pl.ds(sid * rows_per_tile, rows_per_tile)],
            out_hbm.at[cid, pl.ds(sid * rows_per_tile, rows_per_tile)])

    return agg_kernel


def _tc_matmul(x, w, bm=1000):
    n, kin = x.shape
    f = w.shape[1]

    def mm_kernel(x_ref, w_ref, o_ref):
        o_ref[...] = lax.dot_general(
            x_ref[...], w_ref[...], (((1,), (0,)), ((), ())),
            preferred_element_type=jnp.float32)

    return pl.pallas_call(
        mm_kernel,
        grid=(n // bm,),
        in_specs=[pl.BlockSpec((bm, kin), lambda i: (i, 0)),
                  pl.BlockSpec((kin, f), lambda i: (0, 0))],
        out_specs=pl.BlockSpec((bm, f), lambda i: (i, 0)),
        out_shape=jax.ShapeDtypeStruct((n, f), jnp.float32),
    )(x, w)


def _tc_scale_by_dsqo(y, deg):
    n, f = y.shape

    def body(y_ref, d_ref, o_ref):
        d = d_ref[0] + d_ref[1]
        dsqo = lax.rsqrt(jnp.maximum(d[:, 0:1], 1.0))
        o_ref[...] = y_ref[...] * dsqo

    return pl.pallas_call(
        body,
        out_shape=jax.ShapeDtypeStruct((n, f), jnp.float32),
    )(y, deg)


def _tc_relu_rescale(agg_parts, deg, b1):
    _, n, f = agg_parts.shape

    def body(a_ref, d_ref, b_ref, o_ref):
        a = a_ref[0] + a_ref[1]
        d = d_ref[0] + d_ref[1]
        dsqo = lax.rsqrt(jnp.maximum(d[:, 0:1], 1.0))
        dsqi = lax.rsqrt(jnp.maximum(d[:, 1:2], 1.0))
        h = jnp.maximum(a * dsqi + b_ref[...], 0.0)
        o_ref[...] = h * dsqo

    return pl.pallas_call(
        body,
        out_shape=jax.ShapeDtypeStruct((n, f), jnp.float32),
    )(agg_parts, deg, b1.reshape(1, f))


def _tc_final(agg_parts, deg, w2, b2):
    _, n, f = agg_parts.shape
    fo = w2.shape[1]

    def body(a_ref, d_ref, w_ref, b_ref, o_ref):
        a = a_ref[0] + a_ref[1]
        d = d_ref[0] + d_ref[1]
        dsqi = lax.rsqrt(jnp.maximum(d[:, 1:2], 1.0))
        h = a * dsqi
        o_ref[...] = lax.dot_general(
            h, w_ref[...], (((1,), (0,)), ((), ())),
            preferred_element_type=jnp.float32) + b_ref[...]

    return pl.pallas_call(
        body,
        out_shape=jax.ShapeDtypeStruct((n, fo), jnp.float32),
    )(agg_parts, deg, w2, b2.reshape(1, fo))


def kernel(features, edge_index, W1, b1, W2, b2):
    n, _ = features.shape
    e = edge_index.shape[1]
    src = edge_index[0]
    dst = edge_index[1]

    k = e // (NW * CHUNK)
    assert k * NW * CHUNK == e, "edge count must tile into (NW, k, CHUNK)"
    src_r = src.reshape(NW, k, CHUNK)
    dst_r = dst.reshape(NW, k, CHUNK)

    kd = 2 * e // (NW * CHUNK)
    didx = jnp.concatenate([2 * src, 2 * dst + 1]).reshape(NW, kd, CHUNK)

    deg = _make_degree_kernel(n, kd)(didx).reshape(NC, n, 2)
    y = _tc_matmul(features, W1)
    h1 = _tc_scale_by_dsqo(y, deg)

    agg16 = _make_agg_kernel(n, k, 16)
    a1 = agg16(h1, src_r, dst_r)[:, :n, :]
    scaled = _tc_relu_rescale(a1, deg, b1)
    a2 = agg16(scaled, src_r, dst_r)[:, :n, :]
    return _tc_final(a2, deg, W2, b2)
